# baseline (speedup 1.0000x reference)
import functools

import jax
import jax.numpy as jnp
import numpy as np
from jax import lax
from jax.experimental import pallas as pl
from jax.experimental.pallas import tpu as pltpu
from jax.experimental.pallas import tpu_sc as plsc

B = 16
NUM_MODELS = 8
BM = B * NUM_MODELS
V = 40962
NUM_ROIS = 8

_NC = 2
_NS = 16

_TILE = 4096
_VPAD = 45056
_CHUNK = _VPAD // _NS
_SUB = 128
_NSUB = _CHUNK // _SUB
_NVBLK = (V + _TILE - 1) // _TILE - 1
_NBLK = _VPAD // _TILE


def _make_perms():
    rng = np.random.RandomState(0)
    perms = []
    sizes = []
    starts = []
    for side in ("l", "r"):
        labels = rng.randint(0, NUM_ROIS, size=V)
        idx = [np.where(labels == i)[0].astype(np.int32) for i in range(NUM_ROIS)]
        perm = np.zeros(_VPAD, dtype=np.int32)
        st = []
        pos = 0
        for a in idx:
            st.append(pos)
            perm[pos:pos + len(a)] = a
            pos += -(-len(a) // 128) * 128
        assert pos <= _VPAD
        perms.append(perm)
        sizes.append([len(a) for a in idx])
        starts.append(st)
    return np.stack(perms), sizes, starts


_PERMS, _ROI_SIZES, _ROI_STARTS = _make_perms()


def _reduce_body(wm_ref, x_ref, o_ref):
    dn = (((0,), (0,)), ((), ()))
    o_ref[...] = lax.dot_general(x_ref[...], wm_ref[...], dn,
                                 preferred_element_type=jnp.float32)


def _reduce_transposed(data, wmat):
    return pl.pallas_call(
        _reduce_body,
        grid=(_NBLK,),
        in_specs=[
            pl.BlockSpec((BM, B), lambda i: (0, 0)),
            pl.BlockSpec((BM, _TILE), lambda i: (0, jnp.minimum(i, _NVBLK))),
        ],
        out_specs=pl.BlockSpec((_TILE, B), lambda i: (i, 0)),
        out_shape=jax.ShapeDtypeStruct((_VPAD, B), jnp.float32),
    )(wmat, data)


_CHUNK32 = _VPAD // (_NC * _NS)
_NSUB32 = _CHUNK32 // _SUB


def _gather_kernel_body(table_hbm, perm_hbm, out_hbm, idx_v, rows_v, out_v,
                        sem):
    wid = lax.axis_index("s") * _NC + lax.axis_index("c")
    base = wid * _CHUNK32
    pltpu.sync_copy(perm_hbm.at[pl.ds(base, _CHUNK32)], idx_v)

    copies = []
    for j in range(_NSUB32):
        off = j * _SUB
        copies.append(pltpu.async_copy(
            table_hbm.at[idx_v.at[pl.ds(off, _SUB)]],
            rows_v.at[pl.ds(off, _SUB)], sem))

    lanes = jnp.arange(B, dtype=jnp.int32)

    def xpose(c, carry):
        o0 = c * B
        rows = o0 + lanes
        for b in range(B):
            cols = jnp.full((B,), b, dtype=jnp.int32)
            vec = plsc.load_gather(rows_v, [rows, cols])
            out_v[b, pl.ds(o0, B)] = vec
        return carry

    nxp = _SUB // B
    for j in range(_NSUB32):
        copies[j].wait()
        lax.fori_loop(j * nxp, (j + 1) * nxp, xpose, 0)
    pltpu.sync_copy(out_v, out_hbm.at[:, pl.ds(base, _CHUNK32)])


@functools.cache
def _gather_rows_kernel():
    mesh = plsc.VectorSubcoreMesh(
        core_axis_name="c", subcore_axis_name="s",
        num_cores=_NC, num_subcores=_NS)
    return pl.kernel(
        _gather_kernel_body,
        out_type=jax.ShapeDtypeStruct((B, _VPAD), jnp.float32),
        mesh=mesh,
        scratch_types=[
            pltpu.VMEM((_CHUNK32,), jnp.int32),
            pltpu.VMEM((_CHUNK32, B), jnp.float32),
            pltpu.VMEM((B, _CHUNK32), jnp.float32),
            pltpu.SemaphoreType.DMA,
        ],
        compiler_params=pltpu.CompilerParams(
            use_tc_tiling_on_sc=False, needs_layout_passes=False),
    )


def kernel(data_l, data_r, weight):
    w = (weight / jnp.sum(weight)).astype(jnp.float32)
    wmat = jnp.kron(jnp.eye(B, dtype=jnp.float32), w[:, None])
    gather = _gather_rows_kernel()
    avg_l = _reduce_transposed(data_l.reshape(BM, V), wmat)
    full_l = gather(avg_l, jnp.asarray(_PERMS[0]))
    avg_r = _reduce_transposed(data_r.reshape(BM, V), wmat)
    full_r = gather(avg_r, jnp.asarray(_PERMS[1]))
    outs = []
    for s, full in ((0, full_l), (1, full_r)):
        for size, start in zip(_ROI_SIZES[s], _ROI_STARTS[s]):
            outs.append(lax.slice(full, (0, start), (B, start + size)))
    return tuple(outs)


# speedup vs baseline: 1.0096x; 1.0096x over previous
import functools

import jax
import jax.numpy as jnp
import numpy as np
from jax import lax
from jax.experimental import pallas as pl
from jax.experimental.pallas import tpu as pltpu
from jax.experimental.pallas import tpu_sc as plsc

B = 16
NUM_MODELS = 8
BM = B * NUM_MODELS
V = 40962
NUM_ROIS = 8

_NC = 2
_NS = 16

_TILE = 4096
_VPAD = 45056
_CHUNK = _VPAD // _NS
_SUB = 128
_NSUB = _CHUNK // _SUB
_NVBLK = (V + _TILE - 1) // _TILE - 1
_NBLK = _VPAD // _TILE


def _make_perms():
    rng = np.random.RandomState(0)
    perms = []
    sizes = []
    starts = []
    for side in ("l", "r"):
        labels = rng.randint(0, NUM_ROIS, size=V)
        idx = [np.where(labels == i)[0].astype(np.int32) for i in range(NUM_ROIS)]
        perm = np.zeros(_VPAD, dtype=np.int32)
        st = []
        pos = 0
        for a in idx:
            st.append(pos)
            perm[pos:pos + len(a)] = a
            pos += -(-len(a) // 128) * 128
        assert pos <= _VPAD
        perms.append(perm)
        sizes.append([len(a) for a in idx])
        starts.append(st)
    return np.stack(perms), sizes, starts


_PERMS, _ROI_SIZES, _ROI_STARTS = _make_perms()


def _reduce_body(wm_ref, x_ref, o_ref):
    dn = (((0,), (0,)), ((), ()))
    o_ref[...] = lax.dot_general(x_ref[...], wm_ref[...], dn,
                                 preferred_element_type=jnp.float32)


def _reduce_transposed(data, wmat):
    return pl.pallas_call(
        _reduce_body,
        grid=(_NBLK,),
        in_specs=[
            pl.BlockSpec((BM, B), lambda i: (0, 0)),
            pl.BlockSpec((BM, _TILE), lambda i: (0, jnp.minimum(i, _NVBLK))),
        ],
        out_specs=pl.BlockSpec((_TILE, B), lambda i: (i, 0)),
        out_shape=jax.ShapeDtypeStruct((_VPAD, B), jnp.float32),
    )(wmat, data)


_CHUNK32 = _VPAD // (_NC * _NS)
_NSUB32 = _CHUNK32 // _SUB


def _gather_kernel_body(table_hbm, perm_hbm, out_hbm, idx_v, rows_v, out_v,
                        sem, wsem):
    wid = lax.axis_index("s") * _NC + lax.axis_index("c")
    base = wid * _CHUNK32
    pltpu.sync_copy(perm_hbm.at[pl.ds(base, _CHUNK32)], idx_v)

    copies = []
    for j in range(_NSUB32):
        off = j * _SUB
        copies.append(pltpu.async_copy(
            table_hbm.at[idx_v.at[pl.ds(off, _SUB)]],
            rows_v.at[pl.ds(off, _SUB)], sem))

    lanes = jnp.arange(B, dtype=jnp.int32)

    def xpose(c, carry):
        o0 = c * B
        rows = o0 + lanes
        for b in range(B):
            cols = jnp.full((B,), b, dtype=jnp.int32)
            vec = plsc.load_gather(rows_v, [rows, cols])
            out_v[b, pl.ds(o0, B)] = vec
        return carry

    nxp = _SUB // B
    wcopies = []
    for j in range(_NSUB32):
        copies[j].wait()
        lax.fori_loop(j * nxp, (j + 1) * nxp, xpose, 0)
        off = j * _SUB
        wcopies.append(pltpu.async_copy(
            out_v.at[:, pl.ds(off, _SUB)],
            out_hbm.at[:, pl.ds(base + off, _SUB)], wsem))
    for c in wcopies:
        c.wait()


@functools.cache
def _gather_rows_kernel():
    mesh = plsc.VectorSubcoreMesh(
        core_axis_name="c", subcore_axis_name="s",
        num_cores=_NC, num_subcores=_NS)
    return pl.kernel(
        _gather_kernel_body,
        out_type=jax.ShapeDtypeStruct((B, _VPAD), jnp.float32),
        mesh=mesh,
        scratch_types=[
            pltpu.VMEM((_CHUNK32,), jnp.int32),
            pltpu.VMEM((_CHUNK32, B), jnp.float32),
            pltpu.VMEM((B, _CHUNK32), jnp.float32),
            pltpu.SemaphoreType.DMA,
            pltpu.SemaphoreType.DMA,
        ],
        compiler_params=pltpu.CompilerParams(
            use_tc_tiling_on_sc=False, needs_layout_passes=False),
    )


def kernel(data_l, data_r, weight):
    w = (weight / jnp.sum(weight)).astype(jnp.float32)
    wmat = jnp.kron(jnp.eye(B, dtype=jnp.float32), w[:, None])
    gather = _gather_rows_kernel()
    avg_l = _reduce_transposed(data_l.reshape(BM, V), wmat)
    full_l = gather(avg_l, jnp.asarray(_PERMS[0]))
    avg_r = _reduce_transposed(data_r.reshape(BM, V), wmat)
    full_r = gather(avg_r, jnp.asarray(_PERMS[1]))
    outs = []
    for s, full in ((0, full_l), (1, full_r)):
        for size, start in zip(_ROI_SIZES[s], _ROI_STARTS[s]):
            outs.append(lax.slice(full, (0, start), (B, start + size)))
    return tuple(outs)
